# baseline (speedup 1.0000x reference)
import jax
import jax.numpy as jnp
from jax.experimental import pallas as pl
from jax.experimental.pallas import tpu as pltpu

_N = 10000
_BM = 1000
_BK = 1024
_NM = _N // _BM
_NK = (_N + _BK - 1) // _BK
_REM = _N - (_NK - 1) * _BK
_D = 128

_PREC = jax.lax.Precision.HIGHEST


_LANCZOS = (
    676.5203681218851, -1259.1392167224028, 771.32342877765313,
    -176.61502916214059, 12.507343278686905, -0.13857109526572012,
    9.9843695780195716e-6, 1.5056327351493116e-7,
)
_HALF_LOG_2PI = 0.91893853320467274178


def _exp_lgamma(a):
    x = a - 1.0
    z = 0.99999999999980993
    for i, c in enumerate(_LANCZOS):
        z = z + c / (x + (i + 1.0))
    t = x + 7.5
    return jnp.exp(_HALF_LOG_2PI + (x + 0.5) * jnp.log(t) - t + jnp.log(z))


def _proj_kernel(x_ref, w_ref, o_ref):
    o_ref[...] = jnp.dot(x_ref[...], w_ref[...],
                         preferred_element_type=jnp.float32, precision=_PREC)


def _masked_tiles(adj_ref, y_ref):
    ci = jax.lax.broadcasted_iota(jnp.int32, (_BM, _BK), 1)
    a = jnp.where(ci < _REM, adj_ref[...], 0.0)
    ri = jax.lax.broadcasted_iota(jnp.int32, (_BK, _D), 0)
    y = jnp.where(ri < _REM, y_ref[...], 0.0)
    return a, y


def _layer1_kernel(adj_ref, y_ref, o_ref, acc_ref):
    k = pl.program_id(1)

    @pl.when(k == 0)
    def _():
        acc_ref[...] = jnp.zeros_like(acc_ref)

    @pl.when(k < _NK - 1)
    def _():
        acc_ref[...] += jnp.dot(adj_ref[...], y_ref[...],
                                preferred_element_type=jnp.float32,
                                precision=_PREC)

    @pl.when(k == _NK - 1)
    def _():
        a, y = _masked_tiles(adj_ref, y_ref)
        z = acc_ref[...] + jnp.dot(a, y, preferred_element_type=jnp.float32,
                                   precision=_PREC)
        o_ref[...] = jax.nn.softplus(z)


def _layer2_kernel(adj_ref, y_ref, phi_ref, lbd_ref, kap_ref, acc_ref):
    k = pl.program_id(1)

    @pl.when(k == 0)
    def _():
        acc_ref[...] = jnp.zeros_like(acc_ref)

    @pl.when(k < _NK - 1)
    def _():
        acc_ref[...] += jnp.dot(adj_ref[...], y_ref[...],
                                preferred_element_type=jnp.float32,
                                precision=_PREC)

    @pl.when(k == _NK - 1)
    def _():
        a, y = _masked_tiles(adj_ref, y_ref)
        z = acc_ref[...] + jnp.dot(a, y, preferred_element_type=jnp.float32,
                                   precision=_PREC)
        sp = jax.nn.softplus(z)
        lbd = sp[:, :64]
        kap = sp[:, 64:65] + 0.1
        phi = lbd * _exp_lgamma(1.0 + 1.0 / kap)
        phi_ref[...] = phi
        lbd_ref[...] = lbd
        kap_ref[...] = kap


def _proj(x, w):
    return pl.pallas_call(
        _proj_kernel,
        grid=(_NM,),
        in_specs=[
            pl.BlockSpec((_BM, _D), lambda m: (m, 0)),
            pl.BlockSpec((_D, _D), lambda m: (0, 0)),
        ],
        out_specs=pl.BlockSpec((_BM, _D), lambda m: (m, 0)),
        out_shape=jax.ShapeDtypeStruct((_N, _D), jnp.float32),
    )(x, w)


def _layer1(adj, y):
    return pl.pallas_call(
        _layer1_kernel,
        grid=(_NM, _NK),
        in_specs=[
            pl.BlockSpec((_BM, _BK), lambda m, k: (m, k)),
            pl.BlockSpec((_BK, _D), lambda m, k: (k, 0)),
        ],
        out_specs=pl.BlockSpec((_BM, _D), lambda m, k: (m, 0)),
        out_shape=jax.ShapeDtypeStruct((_N, _D), jnp.float32),
        scratch_shapes=[pltpu.VMEM((_BM, _D), jnp.float32)],
        compiler_params=pltpu.CompilerParams(
            dimension_semantics=("parallel", "arbitrary")),
    )(adj, y)


def _layer2(adj, y):
    return pl.pallas_call(
        _layer2_kernel,
        grid=(_NM, _NK),
        in_specs=[
            pl.BlockSpec((_BM, _BK), lambda m, k: (m, k)),
            pl.BlockSpec((_BK, _D), lambda m, k: (k, 0)),
        ],
        out_specs=[
            pl.BlockSpec((_BM, 64), lambda m, k: (m, 0)),
            pl.BlockSpec((_BM, 64), lambda m, k: (m, 0)),
            pl.BlockSpec((_BM, 1), lambda m, k: (m, 0)),
        ],
        out_shape=[
            jax.ShapeDtypeStruct((_N, 64), jnp.float32),
            jax.ShapeDtypeStruct((_N, 64), jnp.float32),
            jax.ShapeDtypeStruct((_N, 1), jnp.float32),
        ],
        scratch_shapes=[pltpu.VMEM((_BM, _D), jnp.float32)],
        compiler_params=pltpu.CompilerParams(
            dimension_semantics=("parallel", "arbitrary")),
    )(adj, y)


def kernel(adj, x, W1, W2):
    W2p = jnp.pad(W2, ((0, 0), (0, _D - W2.shape[1])))
    y1 = _proj(x, W1)
    h1 = _layer1(adj, y1)
    y2 = _proj(h1, W2p)
    phi, lbd, kap = _layer2(adj, y2)
    return (phi, lbd, kap)


# speedup vs baseline: 2.0236x; 2.0236x over previous
import jax
import jax.numpy as jnp
from jax.experimental import pallas as pl
from jax.experimental.pallas import tpu as pltpu

_N = 10000
_BM = 1000
_BK = 1024
_NM = _N // _BM
_NK = (_N + _BK - 1) // _BK
_REM = _N - (_NK - 1) * _BK
_D = 128

_PREC = jax.lax.Precision.DEFAULT


_LANCZOS = (
    676.5203681218851, -1259.1392167224028, 771.32342877765313,
    -176.61502916214059, 12.507343278686905, -0.13857109526572012,
    9.9843695780195716e-6, 1.5056327351493116e-7,
)
_HALF_LOG_2PI = 0.91893853320467274178


def _exp_lgamma(a):
    x = a - 1.0
    z = 0.99999999999980993
    for i, c in enumerate(_LANCZOS):
        z = z + c / (x + (i + 1.0))
    t = x + 7.5
    return jnp.exp(_HALF_LOG_2PI + (x + 0.5) * jnp.log(t) - t + jnp.log(z))


def _proj_kernel(x_ref, w_ref, o_ref):
    o_ref[...] = jnp.dot(x_ref[...], w_ref[...],
                         preferred_element_type=jnp.float32, precision=_PREC)


def _masked_tiles(adj_ref, y_ref):
    ci = jax.lax.broadcasted_iota(jnp.int32, (_BM, _BK), 1)
    a = jnp.where(ci < _REM, adj_ref[...], 0.0)
    ri = jax.lax.broadcasted_iota(jnp.int32, (_BK, _D), 0)
    y = jnp.where(ri < _REM, y_ref[...], 0.0)
    return a, y


def _layer1_kernel(adj_ref, y_ref, o_ref, acc_ref):
    k = pl.program_id(1)

    @pl.when(k == 0)
    def _():
        acc_ref[...] = jnp.zeros_like(acc_ref)

    @pl.when(k < _NK - 1)
    def _():
        acc_ref[...] += jnp.dot(adj_ref[...], y_ref[...],
                                preferred_element_type=jnp.float32,
                                precision=_PREC)

    @pl.when(k == _NK - 1)
    def _():
        a, y = _masked_tiles(adj_ref, y_ref)
        z = acc_ref[...] + jnp.dot(a, y, preferred_element_type=jnp.float32,
                                   precision=_PREC)
        o_ref[...] = jax.nn.softplus(z)


def _layer2_kernel(adj_ref, y_ref, phi_ref, lbd_ref, kap_ref, acc_ref):
    k = pl.program_id(1)

    @pl.when(k == 0)
    def _():
        acc_ref[...] = jnp.zeros_like(acc_ref)

    @pl.when(k < _NK - 1)
    def _():
        acc_ref[...] += jnp.dot(adj_ref[...], y_ref[...],
                                preferred_element_type=jnp.float32,
                                precision=_PREC)

    @pl.when(k == _NK - 1)
    def _():
        a, y = _masked_tiles(adj_ref, y_ref)
        z = acc_ref[...] + jnp.dot(a, y, preferred_element_type=jnp.float32,
                                   precision=_PREC)
        sp = jax.nn.softplus(z)
        lbd = sp[:, :64]
        kap = sp[:, 64:65] + 0.1
        phi = lbd * _exp_lgamma(1.0 + 1.0 / kap)
        phi_ref[...] = phi
        lbd_ref[...] = lbd
        kap_ref[...] = kap


def _proj(x, w):
    return pl.pallas_call(
        _proj_kernel,
        grid=(_NM,),
        in_specs=[
            pl.BlockSpec((_BM, _D), lambda m: (m, 0)),
            pl.BlockSpec((_D, _D), lambda m: (0, 0)),
        ],
        out_specs=pl.BlockSpec((_BM, _D), lambda m: (m, 0)),
        out_shape=jax.ShapeDtypeStruct((_N, _D), jnp.float32),
    )(x, w)


def _layer1(adj, y):
    return pl.pallas_call(
        _layer1_kernel,
        grid=(_NM, _NK),
        in_specs=[
            pl.BlockSpec((_BM, _BK), lambda m, k: (m, k)),
            pl.BlockSpec((_BK, _D), lambda m, k: (k, 0)),
        ],
        out_specs=pl.BlockSpec((_BM, _D), lambda m, k: (m, 0)),
        out_shape=jax.ShapeDtypeStruct((_N, _D), jnp.float32),
        scratch_shapes=[pltpu.VMEM((_BM, _D), jnp.float32)],
        compiler_params=pltpu.CompilerParams(
            dimension_semantics=("parallel", "arbitrary")),
    )(adj, y)


def _layer2(adj, y):
    return pl.pallas_call(
        _layer2_kernel,
        grid=(_NM, _NK),
        in_specs=[
            pl.BlockSpec((_BM, _BK), lambda m, k: (m, k)),
            pl.BlockSpec((_BK, _D), lambda m, k: (k, 0)),
        ],
        out_specs=[
            pl.BlockSpec((_BM, 64), lambda m, k: (m, 0)),
            pl.BlockSpec((_BM, 64), lambda m, k: (m, 0)),
            pl.BlockSpec((_BM, 1), lambda m, k: (m, 0)),
        ],
        out_shape=[
            jax.ShapeDtypeStruct((_N, 64), jnp.float32),
            jax.ShapeDtypeStruct((_N, 64), jnp.float32),
            jax.ShapeDtypeStruct((_N, 1), jnp.float32),
        ],
        scratch_shapes=[pltpu.VMEM((_BM, _D), jnp.float32)],
        compiler_params=pltpu.CompilerParams(
            dimension_semantics=("parallel", "arbitrary")),
    )(adj, y)


def kernel(adj, x, W1, W2):
    W2p = jnp.pad(W2, ((0, 0), (0, _D - W2.shape[1])))
    y1 = _proj(x, W1)
    h1 = _layer1(adj, y1)
    y2 = _proj(h1, W2p)
    phi, lbd, kap = _layer2(adj, y2)
    return (phi, lbd, kap)


# speedup vs baseline: 2.4713x; 1.2213x over previous
import jax
import jax.numpy as jnp
from jax.experimental import pallas as pl
from jax.experimental.pallas import tpu as pltpu

_N = 10000
_BM = 2000
_BK = 2048
_NM = _N // _BM
_NK = (_N + _BK - 1) // _BK
_REM = _N - (_NK - 1) * _BK
_D = 128

_PREC = jax.lax.Precision.DEFAULT


_LANCZOS = (
    676.5203681218851, -1259.1392167224028, 771.32342877765313,
    -176.61502916214059, 12.507343278686905, -0.13857109526572012,
    9.9843695780195716e-6, 1.5056327351493116e-7,
)
_HALF_LOG_2PI = 0.91893853320467274178


def _exp_lgamma(a):
    x = a - 1.0
    z = 0.99999999999980993
    for i, c in enumerate(_LANCZOS):
        z = z + c / (x + (i + 1.0))
    t = x + 7.5
    return jnp.exp(_HALF_LOG_2PI + (x + 0.5) * jnp.log(t) - t + jnp.log(z))


def _proj_kernel(x_ref, w_ref, o_ref):
    o_ref[...] = jnp.dot(x_ref[...], w_ref[...],
                         preferred_element_type=jnp.float32, precision=_PREC)


def _masked_tiles(adj_ref, y_ref):
    ci = jax.lax.broadcasted_iota(jnp.int32, (_BM, _BK), 1)
    a = jnp.where(ci < _REM, adj_ref[...], 0.0)
    ri = jax.lax.broadcasted_iota(jnp.int32, (_BK, _D), 0)
    y = jnp.where(ri < _REM, y_ref[...], 0.0)
    return a, y


def _layer1_kernel(adj_ref, y_ref, o_ref, acc_ref):
    k = pl.program_id(1)

    @pl.when(k == 0)
    def _():
        acc_ref[...] = jnp.zeros_like(acc_ref)

    @pl.when(k < _NK - 1)
    def _():
        acc_ref[...] += jnp.dot(adj_ref[...], y_ref[...],
                                preferred_element_type=jnp.float32,
                                precision=_PREC)

    @pl.when(k == _NK - 1)
    def _():
        a, y = _masked_tiles(adj_ref, y_ref)
        z = acc_ref[...] + jnp.dot(a, y, preferred_element_type=jnp.float32,
                                   precision=_PREC)
        o_ref[...] = jax.nn.softplus(z)


def _layer2_kernel(adj_ref, y_ref, phi_ref, lbd_ref, kap_ref, acc_ref):
    k = pl.program_id(1)

    @pl.when(k == 0)
    def _():
        acc_ref[...] = jnp.zeros_like(acc_ref)

    @pl.when(k < _NK - 1)
    def _():
        acc_ref[...] += jnp.dot(adj_ref[...], y_ref[...],
                                preferred_element_type=jnp.float32,
                                precision=_PREC)

    @pl.when(k == _NK - 1)
    def _():
        a, y = _masked_tiles(adj_ref, y_ref)
        z = acc_ref[...] + jnp.dot(a, y, preferred_element_type=jnp.float32,
                                   precision=_PREC)
        sp = jax.nn.softplus(z)
        lbd = sp[:, :64]
        kap = sp[:, 64:65] + 0.1
        phi = lbd * _exp_lgamma(1.0 + 1.0 / kap)
        phi_ref[...] = phi
        lbd_ref[...] = lbd
        kap_ref[...] = kap


def _proj(x, w):
    return pl.pallas_call(
        _proj_kernel,
        grid=(_NM,),
        in_specs=[
            pl.BlockSpec((_BM, _D), lambda m: (m, 0)),
            pl.BlockSpec((_D, _D), lambda m: (0, 0)),
        ],
        out_specs=pl.BlockSpec((_BM, _D), lambda m: (m, 0)),
        out_shape=jax.ShapeDtypeStruct((_N, _D), jnp.float32),
    )(x, w)


def _layer1(adj, y):
    return pl.pallas_call(
        _layer1_kernel,
        grid=(_NM, _NK),
        in_specs=[
            pl.BlockSpec((_BM, _BK), lambda m, k: (m, k)),
            pl.BlockSpec((_BK, _D), lambda m, k: (k, 0)),
        ],
        out_specs=pl.BlockSpec((_BM, _D), lambda m, k: (m, 0)),
        out_shape=jax.ShapeDtypeStruct((_N, _D), jnp.float32),
        scratch_shapes=[pltpu.VMEM((_BM, _D), jnp.float32)],
        compiler_params=pltpu.CompilerParams(
            dimension_semantics=("parallel", "arbitrary")),
    )(adj, y)


def _layer2(adj, y):
    return pl.pallas_call(
        _layer2_kernel,
        grid=(_NM, _NK),
        in_specs=[
            pl.BlockSpec((_BM, _BK), lambda m, k: (m, k)),
            pl.BlockSpec((_BK, _D), lambda m, k: (k, 0)),
        ],
        out_specs=[
            pl.BlockSpec((_BM, 64), lambda m, k: (m, 0)),
            pl.BlockSpec((_BM, 64), lambda m, k: (m, 0)),
            pl.BlockSpec((_BM, 1), lambda m, k: (m, 0)),
        ],
        out_shape=[
            jax.ShapeDtypeStruct((_N, 64), jnp.float32),
            jax.ShapeDtypeStruct((_N, 64), jnp.float32),
            jax.ShapeDtypeStruct((_N, 1), jnp.float32),
        ],
        scratch_shapes=[pltpu.VMEM((_BM, _D), jnp.float32)],
        compiler_params=pltpu.CompilerParams(
            dimension_semantics=("parallel", "arbitrary")),
    )(adj, y)


def kernel(adj, x, W1, W2):
    W2p = jnp.pad(W2, ((0, 0), (0, _D - W2.shape[1])))
    y1 = _proj(x, W1)
    h1 = _layer1(adj, y1)
    y2 = _proj(h1, W2p)
    phi, lbd, kap = _layer2(adj, y2)
    return (phi, lbd, kap)


# speedup vs baseline: 2.6025x; 1.0531x over previous
import jax
import jax.numpy as jnp
from jax.experimental import pallas as pl
from jax.experimental.pallas import tpu as pltpu

_N = 10000
_BM = 2000
_BK = 2048
_NM = _N // _BM
_NK = (_N + _BK - 1) // _BK
_REM = _N - (_NK - 1) * _BK
_D = 128

_LANCZOS = (
    676.5203681218851, -1259.1392167224028, 771.32342877765313,
    -176.61502916214059, 12.507343278686905, -0.13857109526572012,
    9.9843695780195716e-6, 1.5056327351493116e-7,
)
_HALF_LOG_2PI = 0.91893853320467274178


def _exp_lgamma(a):
    x = a - 1.0
    z = 0.99999999999980993
    for i, c in enumerate(_LANCZOS):
        z = z + c / (x + (i + 1.0))
    t = x + 7.5
    return jnp.exp(_HALF_LOG_2PI + (x + 0.5) * jnp.log(t) - t + jnp.log(z))


def _dot(a, b):
    return jnp.dot(a, b, preferred_element_type=jnp.float32)


def _tile_update(adj_ref, x_ref, w_ref, acc_ref, k):
    @pl.when(k < _NK - 1)
    def _():
        acc_ref[...] += _dot(adj_ref[...], _dot(x_ref[...], w_ref[...]))

    @pl.when(k == _NK - 1)
    def _():
        ci = jax.lax.broadcasted_iota(jnp.int32, (_BM, _BK), 1)
        a = jnp.where(ci < _REM, adj_ref[...], 0.0)
        ri = jax.lax.broadcasted_iota(jnp.int32, (_BK, _D), 0)
        xm = jnp.where(ri < _REM, x_ref[...], 0.0)
        acc_ref[...] += _dot(a, _dot(xm, w_ref[...]))


def _layer1_kernel(adj_ref, x_ref, w_ref, o_ref, acc_ref):
    k = pl.program_id(1)

    @pl.when(k == 0)
    def _():
        acc_ref[...] = jnp.zeros_like(acc_ref)

    _tile_update(adj_ref, x_ref, w_ref, acc_ref, k)

    @pl.when(k == _NK - 1)
    def _():
        o_ref[...] = jax.nn.softplus(acc_ref[...])


def _layer2_kernel(adj_ref, h_ref, w_ref, phi_ref, lbd_ref, kap_ref, acc_ref):
    k = pl.program_id(1)

    @pl.when(k == 0)
    def _():
        acc_ref[...] = jnp.zeros_like(acc_ref)

    _tile_update(adj_ref, h_ref, w_ref, acc_ref, k)

    @pl.when(k == _NK - 1)
    def _():
        sp = jax.nn.softplus(acc_ref[...])
        lbd = sp[:, :64]
        kap = sp[:, 64:65] + 0.1
        phi = lbd * _exp_lgamma(1.0 + 1.0 / kap)
        phi_ref[...] = phi
        lbd_ref[...] = lbd
        kap_ref[...] = kap


def _layer1(adj, x, w):
    return pl.pallas_call(
        _layer1_kernel,
        grid=(_NM, _NK),
        in_specs=[
            pl.BlockSpec((_BM, _BK), lambda m, k: (m, k)),
            pl.BlockSpec((_BK, _D), lambda m, k: (k, 0)),
            pl.BlockSpec((_D, _D), lambda m, k: (0, 0)),
        ],
        out_specs=pl.BlockSpec((_BM, _D), lambda m, k: (m, 0)),
        out_shape=jax.ShapeDtypeStruct((_N, _D), jnp.float32),
        scratch_shapes=[pltpu.VMEM((_BM, _D), jnp.float32)],
        compiler_params=pltpu.CompilerParams(
            dimension_semantics=("parallel", "arbitrary")),
    )(adj, x, w)


def _layer2(adj, h, w):
    return pl.pallas_call(
        _layer2_kernel,
        grid=(_NM, _NK),
        in_specs=[
            pl.BlockSpec((_BM, _BK), lambda m, k: (m, k)),
            pl.BlockSpec((_BK, _D), lambda m, k: (k, 0)),
            pl.BlockSpec((_D, _D), lambda m, k: (0, 0)),
        ],
        out_specs=[
            pl.BlockSpec((_BM, 64), lambda m, k: (m, 0)),
            pl.BlockSpec((_BM, 64), lambda m, k: (m, 0)),
            pl.BlockSpec((_BM, 1), lambda m, k: (m, 0)),
        ],
        out_shape=[
            jax.ShapeDtypeStruct((_N, 64), jnp.float32),
            jax.ShapeDtypeStruct((_N, 64), jnp.float32),
            jax.ShapeDtypeStruct((_N, 1), jnp.float32),
        ],
        scratch_shapes=[pltpu.VMEM((_BM, _D), jnp.float32)],
        compiler_params=pltpu.CompilerParams(
            dimension_semantics=("parallel", "arbitrary")),
    )(adj, h, w)


def kernel(adj, x, W1, W2):
    W2p = jnp.pad(W2, ((0, 0), (0, _D - W2.shape[1])))
    h1 = _layer1(adj, x, W1)
    phi, lbd, kap = _layer2(adj, h1, W2p)
    return (phi, lbd, kap)


# speedup vs baseline: 2.6029x; 1.0002x over previous
import jax
import jax.numpy as jnp
from jax.experimental import pallas as pl
from jax.experimental.pallas import tpu as pltpu

_N = 10000
_BM = 2000
_BK = 2560
_NM = _N // _BM
_NK = (_N + _BK - 1) // _BK
_REM = _N - (_NK - 1) * _BK
_D = 128

_LANCZOS = (
    676.5203681218851, -1259.1392167224028, 771.32342877765313,
    -176.61502916214059, 12.507343278686905, -0.13857109526572012,
    9.9843695780195716e-6, 1.5056327351493116e-7,
)
_HALF_LOG_2PI = 0.91893853320467274178


def _exp_lgamma(a):
    x = a - 1.0
    z = 0.99999999999980993
    for i, c in enumerate(_LANCZOS):
        z = z + c / (x + (i + 1.0))
    t = x + 7.5
    return jnp.exp(_HALF_LOG_2PI + (x + 0.5) * jnp.log(t) - t + jnp.log(z))


def _dot(a, b):
    return jnp.dot(a, b, preferred_element_type=jnp.float32)


def _tile_update(adj_ref, x_ref, w_ref, acc_ref, k):
    @pl.when(k < _NK - 1)
    def _():
        acc_ref[...] += _dot(adj_ref[...], _dot(x_ref[...], w_ref[...]))

    @pl.when(k == _NK - 1)
    def _():
        ci = jax.lax.broadcasted_iota(jnp.int32, (_BM, _BK), 1)
        a = jnp.where(ci < _REM, adj_ref[...], 0.0)
        ri = jax.lax.broadcasted_iota(jnp.int32, (_BK, _D), 0)
        xm = jnp.where(ri < _REM, x_ref[...], 0.0)
        acc_ref[...] += _dot(a, _dot(xm, w_ref[...]))


def _layer1_kernel(adj_ref, x_ref, w_ref, o_ref, acc_ref):
    k = pl.program_id(1)

    @pl.when(k == 0)
    def _():
        acc_ref[...] = jnp.zeros_like(acc_ref)

    _tile_update(adj_ref, x_ref, w_ref, acc_ref, k)

    @pl.when(k == _NK - 1)
    def _():
        o_ref[...] = jax.nn.softplus(acc_ref[...])


def _layer2_kernel(adj_ref, h_ref, w_ref, phi_ref, lbd_ref, kap_ref, acc_ref):
    k = pl.program_id(1)

    @pl.when(k == 0)
    def _():
        acc_ref[...] = jnp.zeros_like(acc_ref)

    _tile_update(adj_ref, h_ref, w_ref, acc_ref, k)

    @pl.when(k == _NK - 1)
    def _():
        sp = jax.nn.softplus(acc_ref[...])
        lbd = sp[:, :64]
        kap = sp[:, 64:65] + 0.1
        phi = lbd * _exp_lgamma(1.0 + 1.0 / kap)
        phi_ref[...] = phi
        lbd_ref[...] = lbd
        kap_ref[...] = kap


def _layer1(adj, x, w):
    return pl.pallas_call(
        _layer1_kernel,
        grid=(_NM, _NK),
        in_specs=[
            pl.BlockSpec((_BM, _BK), lambda m, k: (m, k)),
            pl.BlockSpec((_BK, _D), lambda m, k: (k, 0)),
            pl.BlockSpec((_D, _D), lambda m, k: (0, 0)),
        ],
        out_specs=pl.BlockSpec((_BM, _D), lambda m, k: (m, 0)),
        out_shape=jax.ShapeDtypeStruct((_N, _D), jnp.float32),
        scratch_shapes=[pltpu.VMEM((_BM, _D), jnp.float32)],
        compiler_params=pltpu.CompilerParams(
            dimension_semantics=("parallel", "arbitrary")),
    )(adj, x, w)


def _layer2(adj, h, w):
    return pl.pallas_call(
        _layer2_kernel,
        grid=(_NM, _NK),
        in_specs=[
            pl.BlockSpec((_BM, _BK), lambda m, k: (m, k)),
            pl.BlockSpec((_BK, _D), lambda m, k: (k, 0)),
            pl.BlockSpec((_D, _D), lambda m, k: (0, 0)),
        ],
        out_specs=[
            pl.BlockSpec((_BM, 64), lambda m, k: (m, 0)),
            pl.BlockSpec((_BM, 64), lambda m, k: (m, 0)),
            pl.BlockSpec((_BM, 1), lambda m, k: (m, 0)),
        ],
        out_shape=[
            jax.ShapeDtypeStruct((_N, 64), jnp.float32),
            jax.ShapeDtypeStruct((_N, 64), jnp.float32),
            jax.ShapeDtypeStruct((_N, 1), jnp.float32),
        ],
        scratch_shapes=[pltpu.VMEM((_BM, _D), jnp.float32)],
        compiler_params=pltpu.CompilerParams(
            dimension_semantics=("parallel", "arbitrary")),
    )(adj, h, w)


def kernel(adj, x, W1, W2):
    W2p = jnp.pad(W2, ((0, 0), (0, _D - W2.shape[1])))
    h1 = _layer1(adj, x, W1)
    phi, lbd, kap = _layer2(adj, h1, W2p)
    return (phi, lbd, kap)


# speedup vs baseline: 2.7211x; 1.0454x over previous
import jax
import jax.numpy as jnp
from jax.experimental import pallas as pl
from jax.experimental.pallas import tpu as pltpu

_N = 10000
_BM = 2000
_BK = 2048
_NM = _N // _BM
_NK = (_N + _BK - 1) // _BK
_REM = _N - (_NK - 1) * _BK
_NPAD = _NK * _BK
_D = 128

_LANCZOS = (
    676.5203681218851, -1259.1392167224028, 771.32342877765313,
    -176.61502916214059, 12.507343278686905, -0.13857109526572012,
    9.9843695780195716e-6, 1.5056327351493116e-7,
)
_HALF_LOG_2PI = 0.91893853320467274178


def _exp_lgamma(a):
    x = a - 1.0
    z = 0.99999999999980993
    for i, c in enumerate(_LANCZOS):
        z = z + c / (x + (i + 1.0))
    t = x + 7.5
    return jnp.exp(_HALF_LOG_2PI + (x + 0.5) * jnp.log(t) - t + jnp.log(z))


def _dot(a, b):
    return jnp.dot(a, b, preferred_element_type=jnp.float32)


def _masked_adj(adj_ref):
    ci = jax.lax.broadcasted_iota(jnp.int32, (_BM, _BK), 1)
    return jnp.where(ci < _REM, adj_ref[...], 0.0)


def _fused_kernel(adj_ref, x_ref, w1_ref, w2_ref,
                  phi_ref, lbd_ref, kap_ref,
                  acc_ref, h1s_ref):
    p = pl.program_id(0)
    m = pl.program_id(1)
    k = pl.program_id(2)

    @pl.when(k == 0)
    def _():
        acc_ref[...] = jnp.zeros_like(acc_ref)

    @pl.when(jnp.logical_and(p == 0, k < _NK - 1))
    def _():
        acc_ref[...] += _dot(adj_ref[...], _dot(x_ref[...], w1_ref[...]))

    @pl.when(jnp.logical_and(p == 0, k == _NK - 1))
    def _():
        ri = jax.lax.broadcasted_iota(jnp.int32, (_BK, _D), 0)
        xm = jnp.where(ri < _REM, x_ref[...], 0.0)
        acc = acc_ref[...] + _dot(_masked_adj(adj_ref), _dot(xm, w1_ref[...]))
        h1s_ref[pl.ds(m * _BM, _BM), :] = jax.nn.softplus(acc)

        @pl.when(m == _NM - 1)
        def _():
            h1s_ref[pl.ds(_N, _NPAD - _N), :] = jnp.zeros(
                (_NPAD - _N, _D), jnp.float32)

    @pl.when(jnp.logical_and(p == 1, k < _NK - 1))
    def _():
        h = h1s_ref[pl.ds(k * _BK, _BK), :]
        acc_ref[...] += _dot(adj_ref[...], _dot(h, w2_ref[...]))

    @pl.when(jnp.logical_and(p == 1, k == _NK - 1))
    def _():
        h = h1s_ref[pl.ds(k * _BK, _BK), :]
        acc = acc_ref[...] + _dot(_masked_adj(adj_ref), _dot(h, w2_ref[...]))
        sp = jax.nn.softplus(acc)
        lbd = sp[:, :64]
        kap = sp[:, 64:65] + 0.1
        phi = lbd * _exp_lgamma(1.0 + 1.0 / kap)
        phi_ref[...] = phi
        lbd_ref[...] = lbd
        kap_ref[...] = kap


def kernel(adj, x, W1, W2):
    W2p = jnp.pad(W2, ((0, 0), (0, _D - W2.shape[1])))
    phi, lbd, kap = pl.pallas_call(
        _fused_kernel,
        grid=(2, _NM, _NK),
        in_specs=[
            pl.BlockSpec((_BM, _BK), lambda p, m, k: (m, k)),
            pl.BlockSpec((_BK, _D), lambda p, m, k: (k * (1 - p), 0)),
            pl.BlockSpec((_D, _D), lambda p, m, k: (0, 0)),
            pl.BlockSpec((_D, _D), lambda p, m, k: (0, 0)),
        ],
        out_specs=[
            pl.BlockSpec((_BM, 64), lambda p, m, k: (p * m, 0)),
            pl.BlockSpec((_BM, 64), lambda p, m, k: (p * m, 0)),
            pl.BlockSpec((_BM, 1), lambda p, m, k: (p * m, 0)),
        ],
        out_shape=[
            jax.ShapeDtypeStruct((_N, 64), jnp.float32),
            jax.ShapeDtypeStruct((_N, 64), jnp.float32),
            jax.ShapeDtypeStruct((_N, 1), jnp.float32),
        ],
        scratch_shapes=[
            pltpu.VMEM((_BM, _D), jnp.float32),
            pltpu.VMEM((_NPAD, _D), jnp.float32),
        ],
        compiler_params=pltpu.CompilerParams(
            dimension_semantics=("arbitrary", "arbitrary", "arbitrary")),
    )(adj, x, W1, W2p)
    return (phi, lbd, kap)


# speedup vs baseline: 2.8542x; 1.0489x over previous
import jax
import jax.numpy as jnp
from jax.experimental import pallas as pl
from jax.experimental.pallas import tpu as pltpu

_N = 10000
_BM = 400
_NM = _N // _BM
_D = 128

_LANCZOS = (
    676.5203681218851, -1259.1392167224028, 771.32342877765313,
    -176.61502916214059, 12.507343278686905, -0.13857109526572012,
    9.9843695780195716e-6, 1.5056327351493116e-7,
)
_HALF_LOG_2PI = 0.91893853320467274178


def _exp_lgamma(a):
    x = a - 1.0
    z = 0.99999999999980993
    for i, c in enumerate(_LANCZOS):
        z = z + c / (x + (i + 1.0))
    t = x + 7.5
    return jnp.exp(_HALF_LOG_2PI + (x + 0.5) * jnp.log(t) - t + jnp.log(z))


def _dot(a, b):
    return jnp.dot(a, b, preferred_element_type=jnp.float32)


def _fused_kernel(adj_ref, x_ref, w1_ref, w2_ref,
                  phi_ref, lbd_ref, kap_ref,
                  ys_ref, h1s_ref):
    p = pl.program_id(0)
    m = pl.program_id(1)

    @pl.when(jnp.logical_and(p == 0, m == 0))
    def _():
        ys_ref[...] = _dot(x_ref[...], w1_ref[...])

    @pl.when(jnp.logical_and(p == 1, m == 0))
    def _():
        ys_ref[...] = _dot(h1s_ref[...], w2_ref[...])

    z = _dot(adj_ref[...], ys_ref[...])

    @pl.when(p == 0)
    def _():
        h1s_ref[pl.ds(m * _BM, _BM), :] = jax.nn.softplus(z)

    @pl.when(p == 1)
    def _():
        sp = jax.nn.softplus(z)
        lbd = sp[:, :64]
        kap = sp[:, 64:65] + 0.1
        phi = lbd * _exp_lgamma(1.0 + 1.0 / kap)
        phi_ref[...] = phi
        lbd_ref[...] = lbd
        kap_ref[...] = kap


def kernel(adj, x, W1, W2):
    W2p = jnp.pad(W2, ((0, 0), (0, _D - W2.shape[1])))
    phi, lbd, kap = pl.pallas_call(
        _fused_kernel,
        grid=(2, _NM),
        in_specs=[
            pl.BlockSpec((_BM, _N), lambda p, m: (m, 0)),
            pl.BlockSpec((_N, _D), lambda p, m: (0, 0)),
            pl.BlockSpec((_D, _D), lambda p, m: (0, 0)),
            pl.BlockSpec((_D, _D), lambda p, m: (0, 0)),
        ],
        out_specs=[
            pl.BlockSpec((_BM, 64), lambda p, m: (p * m, 0)),
            pl.BlockSpec((_BM, 64), lambda p, m: (p * m, 0)),
            pl.BlockSpec((_BM, 1), lambda p, m: (p * m, 0)),
        ],
        out_shape=[
            jax.ShapeDtypeStruct((_N, 64), jnp.float32),
            jax.ShapeDtypeStruct((_N, 64), jnp.float32),
            jax.ShapeDtypeStruct((_N, 1), jnp.float32),
        ],
        scratch_shapes=[
            pltpu.VMEM((_N, _D), jnp.float32),
            pltpu.VMEM((_N, _D), jnp.float32),
        ],
        compiler_params=pltpu.CompilerParams(
            dimension_semantics=("arbitrary", "arbitrary")),
    )(adj, x, W1, W2p)
    return (phi, lbd, kap)
